# baseline (speedup 1.0000x reference)
import functools

import jax
import jax.numpy as jnp
from jax import lax
from jax.experimental import pallas as pl
from jax.experimental.pallas import tpu as pltpu
from jax.experimental.pallas import tpu_sc as plsc

VOCAB = 6400
EMB = 64
SEQ = 20
BATCH = 16384
N_ROWS = BATCH * SEQ
NC = 2
NS = 16
NW = NC * NS
ROWS_PER_W = N_ROWS // NW
CHUNK = 512
NCHUNK = ROWS_PER_W // CHUNK


def _sc_gather(table, idx):
    mesh = plsc.VectorSubcoreMesh(core_axis_name="c", subcore_axis_name="s")

    @functools.partial(
        pl.kernel,
        mesh=mesh,
        out_type=jax.ShapeDtypeStruct((N_ROWS, EMB), jnp.float32),
        scratch_types=[
            pltpu.VMEM((ROWS_PER_W,), jnp.int32),
            pltpu.VMEM((CHUNK, EMB), jnp.float32),
            pltpu.VMEM((CHUNK, EMB), jnp.float32),
            pltpu.SemaphoreType.DMA,
            pltpu.SemaphoreType.DMA,
            pltpu.SemaphoreType.DMA,
            pltpu.SemaphoreType.DMA,
        ],
        compiler_params=pltpu.CompilerParams(use_tc_tiling_on_sc=False),
    )
    def k(table_hbm, idx_hbm, out_hbm, idx_v, buf0, buf1, g0, g1, w0, w1):
        wid = lax.axis_index("s") * NC + lax.axis_index("c")
        base = wid * ROWS_PER_W
        pltpu.sync_copy(idx_hbm.at[pl.ds(base, ROWS_PER_W)], idx_v)

        def gat(c, buf, sem):
            off = pl.multiple_of(c * CHUNK, CHUNK)
            return pltpu.async_copy(table_hbm.at[idx_v.at[pl.ds(off, CHUNK)]],
                                    buf, sem)

        def wr(c, buf, sem):
            off = pl.multiple_of(c * CHUNK, CHUNK)
            return pltpu.async_copy(buf, out_hbm.at[pl.ds(base + off, CHUNK)],
                                    sem)

        def wr_drain(c, buf, sem):
            off = pl.multiple_of(c * CHUNK, CHUNK)
            pltpu.make_async_copy(buf, out_hbm.at[pl.ds(base + off, CHUNK)],
                                  sem).wait()

        def body(t, carry):
            a = t * 2
            b = a + 1
            ga = gat(a, buf0, g0)

            @pl.when(t > 0)
            def _():
                wr_drain(a - 1, buf1, w1)

            gb = gat(b, buf1, g1)
            ga.wait()
            wa = wr(a, buf0, w0)
            gb.wait()
            wr(b, buf1, w1)
            wa.wait()
            return carry

        lax.fori_loop(0, NCHUNK // 2, body, 0, unroll=False)
        wr_drain(NCHUNK - 1, buf1, w1)

    return k(table, idx)


TB = 1024


def _mlp_body(x_ref, w1_ref, b1_ref, w2_ref, b2_ref, w3_ref, b3_ref, o_ref):
    x = x_ref[...]
    h = jnp.dot(x, w1_ref[...], preferred_element_type=jnp.float32)
    h = jnp.maximum(h + b1_ref[...], 0.0)
    h = jnp.dot(h, w2_ref[...], preferred_element_type=jnp.float32)
    h = jnp.maximum(h + b2_ref[...], 0.0)
    o = jnp.dot(h, w3_ref[...], preferred_element_type=jnp.float32)
    o_ref[...] = o + b3_ref[...]


def _mlp(x, W1, b1, W2, b2, W3, b3):
    flat = SEQ * EMB
    grid = (BATCH // TB,)
    return pl.pallas_call(
        _mlp_body,
        grid=grid,
        in_specs=[
            pl.BlockSpec((TB, flat), lambda i: (i, 0)),
            pl.BlockSpec((flat, 128), lambda i: (0, 0)),
            pl.BlockSpec((1, 128), lambda i: (0, 0)),
            pl.BlockSpec((128, 64), lambda i: (0, 0)),
            pl.BlockSpec((1, 64), lambda i: (0, 0)),
            pl.BlockSpec((64, 2), lambda i: (0, 0)),
            pl.BlockSpec((1, 2), lambda i: (0, 0)),
        ],
        out_specs=pl.BlockSpec((TB, 2), lambda i: (i, 0)),
        out_shape=jax.ShapeDtypeStruct((BATCH, 2), jnp.float32),
    )(x, W1, b1, W2, b2, W3, b3)


def kernel(inputs, table, W1, b1, W2, b2, W3, b3):
    idx = inputs.reshape(-1).astype(jnp.int32)
    x = _sc_gather(table, idx)
    x = x.reshape(BATCH, SEQ * EMB)
    return _mlp(x, W1, b1.reshape(1, -1), W2, b2.reshape(1, -1),
                W3, b3.reshape(1, -1))


# speedup vs baseline: 1.5126x; 1.5126x over previous
import functools

import jax
import jax.numpy as jnp
import numpy as np
from jax import lax
from jax.experimental import pallas as pl
from jax.experimental.pallas import tpu as pltpu
from jax.experimental.pallas import tpu_sc as plsc

VOCAB = 6400
EMB = 64
HALF = EMB // 2
SEQ = 20
BATCH = 16384
N_ROWS = BATCH * SEQ
NC = 2
NS = 16
NW = NC * NS
ROWS_PER_W = N_ROWS // NW
CHUNK = 1024
NCHUNK = ROWS_PER_W // CHUNK

_PERM_LO = np.arange(SEQ * EMB).reshape(SEQ, EMB)[:, :HALF].reshape(-1)
_PERM_HI = np.arange(SEQ * EMB).reshape(SEQ, EMB)[:, HALF:].reshape(-1)


def _sc_gather(table_packed, idx):
    mesh = plsc.VectorSubcoreMesh(core_axis_name="c", subcore_axis_name="s")

    @functools.partial(
        pl.kernel,
        mesh=mesh,
        out_type=jax.ShapeDtypeStruct((N_ROWS, HALF), jnp.int32),
        scratch_types=[
            pltpu.VMEM((ROWS_PER_W,), jnp.int32),
            pltpu.VMEM((CHUNK, HALF), jnp.int32),
            pltpu.VMEM((CHUNK, HALF), jnp.int32),
            pltpu.SemaphoreType.DMA,
            pltpu.SemaphoreType.DMA,
            pltpu.SemaphoreType.DMA,
            pltpu.SemaphoreType.DMA,
        ],
        compiler_params=pltpu.CompilerParams(use_tc_tiling_on_sc=False),
    )
    def k(table_hbm, idx_hbm, out_hbm, idx_v, buf0, buf1, g0, g1, w0, w1):
        wid = lax.axis_index("s") * NC + lax.axis_index("c")
        base = wid * ROWS_PER_W
        pltpu.sync_copy(idx_hbm.at[pl.ds(base, ROWS_PER_W)], idx_v)

        def gat(c, buf, sem):
            off = pl.multiple_of(c * CHUNK, CHUNK)
            return pltpu.async_copy(table_hbm.at[idx_v.at[pl.ds(off, CHUNK)]],
                                    buf, sem)

        def wr(c, buf, sem):
            off = pl.multiple_of(c * CHUNK, CHUNK)
            return pltpu.async_copy(buf, out_hbm.at[pl.ds(base + off, CHUNK)],
                                    sem)

        def wr_drain(c, buf, sem):
            off = pl.multiple_of(c * CHUNK, CHUNK)
            pltpu.make_async_copy(buf, out_hbm.at[pl.ds(base + off, CHUNK)],
                                  sem).wait()

        def body(t, carry):
            a = t * 2
            b = a + 1
            ga = gat(a, buf0, g0)

            @pl.when(t > 0)
            def _():
                wr_drain(a - 1, buf1, w1)

            gb = gat(b, buf1, g1)
            ga.wait()
            wa = wr(a, buf0, w0)
            gb.wait()
            wr(b, buf1, w1)
            wa.wait()
            return carry

        lax.fori_loop(0, NCHUNK // 2, body, 0, unroll=False)
        wr_drain(NCHUNK - 1, buf1, w1)

    return k(table_packed, idx)


TB = 1024


def _mlp_body(x_ref, w1a_ref, w1b_ref, b1_ref, w2_ref, b2_ref, w3_ref,
              b3_ref, o_ref):
    xi = x_ref[...]
    xa = lax.bitcast_convert_type(xi << 16, jnp.float32)
    xb = lax.bitcast_convert_type(xi & jnp.int32(-65536), jnp.float32)
    h = jnp.dot(xa, w1a_ref[...], preferred_element_type=jnp.float32)
    h += jnp.dot(xb, w1b_ref[...], preferred_element_type=jnp.float32)
    h = jnp.maximum(h + b1_ref[...], 0.0)
    h = jnp.dot(h, w2_ref[...], preferred_element_type=jnp.float32)
    h = jnp.maximum(h + b2_ref[...], 0.0)
    o = jnp.dot(h, w3_ref[...], preferred_element_type=jnp.float32)
    o_ref[...] = o + b3_ref[...]


def _mlp(x, W1a, W1b, b1, W2, b2, W3, b3):
    flat = SEQ * HALF
    grid = (BATCH // TB,)
    return pl.pallas_call(
        _mlp_body,
        grid=grid,
        in_specs=[
            pl.BlockSpec((TB, flat), lambda i: (i, 0)),
            pl.BlockSpec((flat, 128), lambda i: (0, 0)),
            pl.BlockSpec((flat, 128), lambda i: (0, 0)),
            pl.BlockSpec((1, 128), lambda i: (0, 0)),
            pl.BlockSpec((128, 64), lambda i: (0, 0)),
            pl.BlockSpec((1, 64), lambda i: (0, 0)),
            pl.BlockSpec((64, 2), lambda i: (0, 0)),
            pl.BlockSpec((1, 2), lambda i: (0, 0)),
        ],
        out_specs=pl.BlockSpec((TB, 2), lambda i: (i, 0)),
        out_shape=jax.ShapeDtypeStruct((BATCH, 2), jnp.float32),
    )(x, W1a, W1b, b1, W2, b2, W3, b3)


def kernel(inputs, table, W1, b1, W2, b2, W3, b3):
    idx = inputs.reshape(-1).astype(jnp.int32)
    tb = table.astype(jnp.bfloat16)
    lo = lax.bitcast_convert_type(tb[:, :HALF], jnp.uint16).astype(jnp.uint32)
    hi = lax.bitcast_convert_type(tb[:, HALF:], jnp.uint16).astype(jnp.uint32)
    packed = lax.bitcast_convert_type((hi << 16) | lo, jnp.int32)
    x = _sc_gather(packed, idx)
    x = x.reshape(BATCH, SEQ * HALF)
    W1a = W1[_PERM_LO]
    W1b = W1[_PERM_HI]
    return _mlp(x, W1a, W1b, b1.reshape(1, -1), W2, b2.reshape(1, -1),
                W3, b3.reshape(1, -1))


# speedup vs baseline: 1.6567x; 1.0953x over previous
import functools

import jax
import jax.numpy as jnp
import numpy as np
from jax import lax
from jax.experimental import pallas as pl
from jax.experimental.pallas import tpu as pltpu
from jax.experimental.pallas import tpu_sc as plsc

VOCAB = 6400
EMB = 64
HALF = EMB // 2
SEQ = 20
BATCH = 16384
N_ROWS = BATCH * SEQ
NC = 2
NS = 16
NW = NC * NS
ROWS_PER_W = N_ROWS // NW
CHUNK = 1024
NCHUNK = ROWS_PER_W // CHUNK

_PERM_LO = np.arange(SEQ * EMB).reshape(SEQ, EMB)[:, :HALF].reshape(-1)
_PERM_HI = np.arange(SEQ * EMB).reshape(SEQ, EMB)[:, HALF:].reshape(-1)


def _sc_gather(table_packed, idx):
    mesh = plsc.VectorSubcoreMesh(core_axis_name="c", subcore_axis_name="s")

    @functools.partial(
        pl.kernel,
        mesh=mesh,
        out_type=jax.ShapeDtypeStruct((N_ROWS, HALF), jnp.int32),
        scratch_types=[
            pltpu.VMEM((ROWS_PER_W,), jnp.int32),
            pltpu.VMEM((CHUNK, HALF), jnp.int32),
            pltpu.VMEM((CHUNK, HALF), jnp.int32),
            pltpu.VMEM_SHARED((VOCAB, HALF), jnp.int32),
            pltpu.SemaphoreType.DMA,
            pltpu.SemaphoreType.DMA,
            pltpu.SemaphoreType.DMA,
            pltpu.SemaphoreType.DMA,
        ],
        compiler_params=pltpu.CompilerParams(use_tc_tiling_on_sc=False),
    )
    def k(table_hbm, idx_hbm, out_hbm, idx_v, buf0, buf1, spm_tab,
          g0, g1, w0, w1):
        wid = lax.axis_index("s") * NC + lax.axis_index("c")
        base = wid * ROWS_PER_W

        @pl.when(lax.axis_index("s") == 0)
        def _():
            pltpu.sync_copy(table_hbm, spm_tab)

        pltpu.sync_copy(idx_hbm.at[pl.ds(base, ROWS_PER_W)], idx_v)
        plsc.subcore_barrier()

        def gat(c, buf, sem):
            off = pl.multiple_of(c * CHUNK, CHUNK)
            return pltpu.async_copy(spm_tab.at[idx_v.at[pl.ds(off, CHUNK)]],
                                    buf, sem)

        def wr(c, buf, sem):
            off = pl.multiple_of(c * CHUNK, CHUNK)
            return pltpu.async_copy(buf, out_hbm.at[pl.ds(base + off, CHUNK)],
                                    sem)

        def wr_drain(c, buf, sem):
            off = pl.multiple_of(c * CHUNK, CHUNK)
            pltpu.make_async_copy(buf, out_hbm.at[pl.ds(base + off, CHUNK)],
                                  sem).wait()

        def body(t, carry):
            a = t * 2
            b = a + 1
            ga = gat(a, buf0, g0)

            @pl.when(t > 0)
            def _():
                wr_drain(a - 1, buf1, w1)

            gb = gat(b, buf1, g1)
            ga.wait()
            wa = wr(a, buf0, w0)
            gb.wait()
            wr(b, buf1, w1)
            wa.wait()
            return carry

        lax.fori_loop(0, NCHUNK // 2, body, 0, unroll=False)
        wr_drain(NCHUNK - 1, buf1, w1)

    return k(table_packed, idx)


TB = 1024


def _mlp_body(x_ref, w1a_ref, w1b_ref, b1_ref, w2_ref, b2_ref, w3_ref,
              b3_ref, o_ref):
    xi = x_ref[...]
    xa = lax.bitcast_convert_type(xi << 16, jnp.float32)
    xb = lax.bitcast_convert_type(xi & jnp.int32(-65536), jnp.float32)
    h = jnp.dot(xa, w1a_ref[...], preferred_element_type=jnp.float32)
    h += jnp.dot(xb, w1b_ref[...], preferred_element_type=jnp.float32)
    h = jnp.maximum(h + b1_ref[...], 0.0)
    h = jnp.dot(h, w2_ref[...], preferred_element_type=jnp.float32)
    h = jnp.maximum(h + b2_ref[...], 0.0)
    o = jnp.dot(h, w3_ref[...], preferred_element_type=jnp.float32)
    o_ref[...] = o + b3_ref[...]


def _mlp(x, W1a, W1b, b1, W2, b2, W3, b3):
    flat = SEQ * HALF
    grid = (BATCH // TB,)
    return pl.pallas_call(
        _mlp_body,
        grid=grid,
        in_specs=[
            pl.BlockSpec((TB, flat), lambda i: (i, 0)),
            pl.BlockSpec((flat, 128), lambda i: (0, 0)),
            pl.BlockSpec((flat, 128), lambda i: (0, 0)),
            pl.BlockSpec((1, 128), lambda i: (0, 0)),
            pl.BlockSpec((128, 64), lambda i: (0, 0)),
            pl.BlockSpec((1, 64), lambda i: (0, 0)),
            pl.BlockSpec((64, 2), lambda i: (0, 0)),
            pl.BlockSpec((1, 2), lambda i: (0, 0)),
        ],
        out_specs=pl.BlockSpec((TB, 2), lambda i: (i, 0)),
        out_shape=jax.ShapeDtypeStruct((BATCH, 2), jnp.float32),
    )(x, W1a, W1b, b1, W2, b2, W3, b3)


def kernel(inputs, table, W1, b1, W2, b2, W3, b3):
    idx = inputs.reshape(-1).astype(jnp.int32)
    tb = table.astype(jnp.bfloat16)
    lo = lax.bitcast_convert_type(tb[:, :HALF], jnp.uint16).astype(jnp.uint32)
    hi = lax.bitcast_convert_type(tb[:, HALF:], jnp.uint16).astype(jnp.uint32)
    packed = lax.bitcast_convert_type((hi << 16) | lo, jnp.int32)
    x = _sc_gather(packed, idx)
    x = x.reshape(BATCH, SEQ * HALF)
    W1a = W1[_PERM_LO]
    W1b = W1[_PERM_HI]
    return _mlp(x, W1a, W1b, b1.reshape(1, -1), W2, b2.reshape(1, -1),
                W3, b3.reshape(1, -1))
